# Initial kernel scaffold; baseline (speedup 1.0000x reference)
import jax
import jax.numpy as jnp
from jax.experimental import pallas as pl


def kernel(node_w, node_b, wvf, bvf, wga, w1ab, b1, w2, b2, node_features, adj_matrix, edge_weights):
    raise NotImplementedError("write your pallas kernel here")


# speedup vs baseline: 3.8559x; 3.8559x over previous
import functools

import jax
import jax.numpy as jnp
from jax.experimental import pallas as pl
from jax.experimental.pallas import tpu as pltpu


def _graph_kernel(D, H, L, RB,
                  nf_ref, adj_ref, ew_ref,
                  node_w_ref, node_b_ref,
                  wvf_ref, bvf_ref, wga_ref,
                  w1ab_ref, b1_ref, w2bc_ref, b2_ref,
                  logits_ref, loss_ref,
                  a_scr, bt_scr):
    N = adj_ref.shape[0]

    nf = nf_ref[...]
    h = (nf[:, 0:1] * node_w_ref[0:1, :]
         + nf[:, 1:2] * node_w_ref[1:2, :]
         + nf[:, 2:3] * node_w_ref[2:3, :]
         + node_b_ref[...])

    adj = adj_ref[...]
    for l in range(L):
        vp = jnp.dot(h, wvf_ref[l], preferred_element_type=jnp.float32) + bvf_ref[l]
        agg = jnp.dot(adj, vp[:, :D], preferred_element_type=jnp.float32)
        gate = jax.nn.sigmoid(
            vp[:, D:] + jnp.dot(agg, wga_ref[l], preferred_element_type=jnp.float32))
        h = jnp.maximum(h + gate * agg, 0.0)

    ab = jnp.dot(h, w1ab_ref[...], preferred_element_type=jnp.float32)
    a_scr[...] = ab[:, :H] + b1_ref[...]
    bt_scr[...] = jnp.transpose(ab[:, H:])

    bt = bt_scr[...]
    w2bc = w2bc_ref[...]
    b2 = b2_ref[0]

    def blk(i, acc):
        r0 = pl.multiple_of(i * RB, RB)
        at_blk = jnp.transpose(a_scr[pl.ds(r0, RB), :])
        rows = []
        for s in range(RB):
            acol = at_blk[:, s:s + 1]
            hid = jnp.maximum(acol + bt, 0.0)
            rows.append(jnp.sum(hid * w2bc, axis=0, keepdims=True))
        lg = jnp.concatenate(rows, axis=0) + b2
        logits_ref[pl.ds(r0, RB), :] = lg
        d = lg * adj_ref[pl.ds(r0, RB), :] - ew_ref[pl.ds(r0, RB), :]
        return acc + jnp.sum(d * d)

    sq = jax.lax.fori_loop(0, N // RB, blk, jnp.zeros((1, 1), jnp.float32))
    loss_ref[...] = sq * (1.0 / float(N * N))


def kernel(node_w, node_b, wvf, bvf, wga, w1ab, b1, w2, b2,
           node_features, adj_matrix, edge_weights):
    B, N, _ = node_features.shape
    D = node_w.shape[1]
    L = wvf.shape[0]
    H = b1.shape[1]
    RB = 8
    ew = edge_weights[..., 0]
    w2bc = jnp.broadcast_to(jnp.reshape(w2, (H, 1)), (H, N))

    body = functools.partial(_graph_kernel, D, H, L, RB)

    def per_graph(shape):
        nd = len(shape)
        return pl.BlockSpec((None,) + shape, lambda b, _nd=nd: (b,) + (0,) * _nd)

    def resident(shape):
        nd = len(shape)
        return pl.BlockSpec(shape, lambda b, _nd=nd: (0,) * _nd)

    logits, loss = pl.pallas_call(
        body,
        out_shape=(jax.ShapeDtypeStruct((B, N, N), jnp.float32),
                   jax.ShapeDtypeStruct((B, 1, 1), jnp.float32)),
        grid=(B,),
        in_specs=[
            per_graph((N, 3)),
            per_graph((N, N)),
            per_graph((N, N)),
            resident((3, D)), resident((1, D)),
            resident((L, D, 2 * D)), resident((L, 1, 2 * D)),
            resident((L, D, D)),
            resident((D, 2 * H)), resident((1, H)),
            resident((H, N)),
            pl.BlockSpec(memory_space=pltpu.MemorySpace.SMEM),
        ],
        out_specs=(per_graph((N, N)), per_graph((1, 1))),
        scratch_shapes=[pltpu.VMEM((N, H), jnp.float32),
                        pltpu.VMEM((H, N), jnp.float32)],
        compiler_params=pltpu.CompilerParams(
            dimension_semantics=("parallel",)),
    )(node_features, adj_matrix, ew,
      node_w, node_b, wvf, bvf, wga, w1ab, b1, w2bc, b2)
    return logits, loss.reshape(B)
